# baseline (speedup 1.0000x reference)
import jax
import jax.numpy as jnp
from jax.experimental import pallas as pl
from jax.experimental.pallas import tpu as pltpu

_ALPHA = 0.2
_NEG = -9e15


def _gat_block_kernel(nf_ref, nft_ref, adj_ref, wh_ref, w_ref, a1_ref,
                      a2t_ref, bh_ref, bht_ref, out_ref, projh_ref, tt_ref,
                      projb_ref, *, num_heads, c_head, c_in, n_nodes, blk):
    i = pl.program_id(0)

    @pl.when(i == 0)
    def _init():
        nf = nf_ref[...]
        nft = nft_ref[...]
        ones_col = jnp.ones((n_nodes, 1), dtype=jnp.float32)
        t_rows = []
        for h in range(num_heads):
            wh = wh_ref[pl.ds(h * c_in, c_in), :]
            ph = jnp.dot(nf, wh, preferred_element_type=jnp.float32)
            ph = ph + bh_ref[h:h + 1, :]
            ph_aug = jnp.concatenate([ph, ones_col], axis=1)
            projh_ref[pl.ds(h * n_nodes, n_nodes), :] = ph_aug
            projb_ref[pl.ds(h * n_nodes, n_nodes), :] = ph_aug.astype(
                jnp.bfloat16)
            wrow = w_ref[pl.ds(h * c_head, c_head), :]
            pt = jnp.dot(wrow, nft, preferred_element_type=jnp.float32)
            pt = pt + bht_ref[pl.ds(h * c_head, c_head), :]
            a2 = a2t_ref[pl.ds(h * c_head, c_head), :]
            t_rows.append(jnp.sum(pt * a2, axis=0, keepdims=True))
        for h in range(num_heads):
            t_rows.append(_ALPHA * t_rows[h])
        t_tiles = [jnp.broadcast_to(r, (8, n_nodes)) for r in t_rows]
        tt_ref[...] = jnp.concatenate(t_tiles, axis=0)

    maskf = adj_ref[...].astype(jnp.float32)
    ones_row = jnp.ones((1, n_nodes), dtype=jnp.float32)
    outs = []
    for h in range(num_heads):
        ph_blk = projh_ref[pl.ds(h * n_nodes + i * blk, blk), :]
        a1 = a1_ref[h:h + 1, :]
        s_blk = jnp.sum(ph_blk * a1, axis=1, keepdims=True)
        t8 = tt_ref[pl.ds(8 * h, 8), :]
        ta8 = tt_ref[pl.ds(8 * (num_heads + h), 8), :]
        mt = jnp.max(t8[0:1, :])
        m_i = jnp.maximum(s_blk + mt, 0.0)
        s1 = (s_blk - m_i).reshape(blk // 8, 8, 1)
        s2 = (_ALPHA * s_blk - m_i).reshape(blk // 8, 8, 1)
        za = s1 + t8[None]
        zb = s2 + ta8[None]
        e = jnp.exp(jnp.maximum(za, zb)).reshape(blk, n_nodes) * maskf
        eb = e.astype(jnp.bfloat16)
        ph_full = projb_ref[pl.ds(h * n_nodes, n_nodes), :]
        acc = jnp.dot(eb, ph_full, preferred_element_type=jnp.float32)
        outs.append(acc[:, :c_head] / acc[:, c_head:c_head + 1])
    out_ref[...] = jnp.concatenate(outs, axis=1)


def kernel(node_feats, adj_matrix, W, b, a):
    batch, n_nodes, c_in = node_feats.shape
    num_heads, two_c = a.shape
    c_head = two_c // 2
    c_out = num_heads * c_head

    blk = min(256, n_nodes)
    grid = (n_nodes // blk,)

    wh = jnp.transpose(W.reshape(num_heads, c_head, c_in), (0, 2, 1))
    wh = wh.reshape(num_heads * c_in, c_head)
    a1 = jnp.pad(a[:, :c_head], ((0, 0), (0, 1)))
    a2t = a[:, c_head:].reshape(num_heads * c_head, 1)
    bh = b.reshape(num_heads, c_head)
    bht = b.reshape(num_heads * c_head, 1)

    const = lambda i: (0, 0)
    call = pl.pallas_call(
        lambda *refs: _gat_block_kernel(
            *refs, num_heads=num_heads, c_head=c_head, c_in=c_in,
            n_nodes=n_nodes, blk=blk),
        grid=grid,
        in_specs=[
            pl.BlockSpec((n_nodes, c_in), const),
            pl.BlockSpec((c_in, n_nodes), const),
            pl.BlockSpec((blk, n_nodes), lambda i: (i, 0)),
            pl.BlockSpec((num_heads * c_in, c_head), const),
            pl.BlockSpec((num_heads * c_head, c_in), const),
            pl.BlockSpec((num_heads, c_head + 1), const),
            pl.BlockSpec((num_heads * c_head, 1), const),
            pl.BlockSpec((num_heads, c_head), const),
            pl.BlockSpec((num_heads * c_head, 1), const),
        ],
        out_specs=pl.BlockSpec((blk, c_out), lambda i: (i, 0)),
        out_shape=jax.ShapeDtypeStruct((n_nodes, c_out), jnp.float32),
        scratch_shapes=[
            pltpu.VMEM((num_heads * n_nodes, c_head + 1), jnp.float32),
            pltpu.VMEM((16 * num_heads, n_nodes), jnp.float32),
            pltpu.VMEM((num_heads * n_nodes, c_head + 1), jnp.bfloat16),
        ],
    )

    outs = []
    for bb in range(batch):
        nf = node_feats[bb]
        outs.append(call(nf, nf.T, adj_matrix[bb], wh, W, a1, a2t, bh, bht))
    return jnp.stack(outs, axis=0)


# speedup vs baseline: 1.0946x; 1.0946x over previous
import jax
import jax.numpy as jnp
from jax.experimental import pallas as pl
from jax.experimental.pallas import tpu as pltpu

_ALPHA = 0.2
_NEG = -9e15


def _gat_block_kernel(nf_ref, nft_ref, adj_ref, wh_ref, w_ref, a1_ref,
                      a2t_ref, bh_ref, bht_ref, out_ref, projh_ref, tt_ref,
                      *, num_heads, c_head, c_in, n_nodes, blk):
    i = pl.program_id(0)

    @pl.when(i == 0)
    def _init():
        nf = nf_ref[...]
        nft = nft_ref[...]
        ones_col = jnp.ones((n_nodes, 1), dtype=jnp.float32)
        t_rows = []
        for h in range(num_heads):
            wh = wh_ref[pl.ds(h * c_in, c_in), :]
            ph = jnp.dot(nf, wh, preferred_element_type=jnp.float32)
            ph = ph + bh_ref[h:h + 1, :]
            projh_ref[pl.ds(h * n_nodes, n_nodes), :] = jnp.concatenate(
                [ph, ones_col], axis=1)
            wrow = w_ref[pl.ds(h * c_head, c_head), :]
            pt = jnp.dot(wrow, nft, preferred_element_type=jnp.float32)
            pt = pt + bht_ref[pl.ds(h * c_head, c_head), :]
            a2 = a2t_ref[pl.ds(h * c_head, c_head), :]
            t_rows.append(jnp.sum(pt * a2, axis=0, keepdims=True))
        for h in range(num_heads):
            t_rows.append(_ALPHA * t_rows[h])
        t_tiles = [jnp.broadcast_to(r, (8, n_nodes)) for r in t_rows]
        tt_ref[...] = jnp.concatenate(t_tiles, axis=0)

    maskf = adj_ref[...].astype(jnp.float32)
    ones_row = jnp.ones((1, n_nodes), dtype=jnp.float32)
    outs = []
    for h in range(num_heads):
        ph_blk = projh_ref[pl.ds(h * n_nodes + i * blk, blk), :]
        a1 = a1_ref[h:h + 1, :]
        s_blk = jnp.sum(ph_blk * a1, axis=1, keepdims=True)
        t8 = tt_ref[pl.ds(8 * h, 8), :]
        ta8 = tt_ref[pl.ds(8 * (num_heads + h), 8), :]
        mt = jnp.max(t8[0:1, :])
        m_i = jnp.maximum(s_blk + mt, 0.0)
        s1 = (s_blk - m_i).reshape(blk // 8, 8, 1)
        s2 = (_ALPHA * s_blk - m_i).reshape(blk // 8, 8, 1)
        za = s1 + t8[None]
        zb = s2 + ta8[None]
        e = jnp.exp(jnp.maximum(za, zb)).reshape(blk, n_nodes) * maskf
        ph_full = projh_ref[pl.ds(h * n_nodes, n_nodes), :]
        acc = jnp.dot(e, ph_full, preferred_element_type=jnp.float32)
        outs.append(acc[:, :c_head] / acc[:, c_head:c_head + 1])
    out_ref[...] = jnp.concatenate(outs, axis=1)


def kernel(node_feats, adj_matrix, W, b, a):
    batch, n_nodes, c_in = node_feats.shape
    num_heads, two_c = a.shape
    c_head = two_c // 2
    c_out = num_heads * c_head

    blk = min(512, n_nodes)
    grid = (n_nodes // blk,)

    wh = jnp.transpose(W.reshape(num_heads, c_head, c_in), (0, 2, 1))
    wh = wh.reshape(num_heads * c_in, c_head)
    a1 = jnp.pad(a[:, :c_head], ((0, 0), (0, 1)))
    a2t = a[:, c_head:].reshape(num_heads * c_head, 1)
    bh = b.reshape(num_heads, c_head)
    bht = b.reshape(num_heads * c_head, 1)

    const = lambda i: (0, 0)
    call = pl.pallas_call(
        lambda *refs: _gat_block_kernel(
            *refs, num_heads=num_heads, c_head=c_head, c_in=c_in,
            n_nodes=n_nodes, blk=blk),
        grid=grid,
        in_specs=[
            pl.BlockSpec((n_nodes, c_in), const),
            pl.BlockSpec((c_in, n_nodes), const),
            pl.BlockSpec((blk, n_nodes), lambda i: (i, 0)),
            pl.BlockSpec((num_heads * c_in, c_head), const),
            pl.BlockSpec((num_heads * c_head, c_in), const),
            pl.BlockSpec((num_heads, c_head + 1), const),
            pl.BlockSpec((num_heads * c_head, 1), const),
            pl.BlockSpec((num_heads, c_head), const),
            pl.BlockSpec((num_heads * c_head, 1), const),
        ],
        out_specs=pl.BlockSpec((blk, c_out), lambda i: (i, 0)),
        out_shape=jax.ShapeDtypeStruct((n_nodes, c_out), jnp.float32),
        scratch_shapes=[
            pltpu.VMEM((num_heads * n_nodes, c_head + 1), jnp.float32),
            pltpu.VMEM((16 * num_heads, n_nodes), jnp.float32),
        ],
    )

    outs = []
    for bb in range(batch):
        nf = node_feats[bb]
        outs.append(call(nf, nf.T, adj_matrix[bb], wh, W, a1, a2t, bh, bht))
    return jnp.stack(outs, axis=0)


# speedup vs baseline: 1.0948x; 1.0002x over previous
import jax
import jax.numpy as jnp
from jax.experimental import pallas as pl
from jax.experimental.pallas import tpu as pltpu

_ALPHA = 0.2
_NEG = -9e15


def _gat_block_kernel(nf_ref, nft_ref, adj_ref, wh_ref, w_ref, a1_ref,
                      a2t_ref, bh_ref, bht_ref, out_ref, projh_ref, tt_ref,
                      *, num_heads, c_head, c_in, n_nodes, blk):
    i = pl.program_id(0)

    @pl.when(i == 0)
    def _init():
        nf = nf_ref[...]
        nft = nft_ref[...]
        ones_col = jnp.ones((n_nodes, 1), dtype=jnp.float32)
        t_rows = []
        for h in range(num_heads):
            wh = wh_ref[pl.ds(h * c_in, c_in), :]
            ph = jnp.dot(nf, wh, preferred_element_type=jnp.float32)
            ph = ph + bh_ref[h:h + 1, :]
            projh_ref[pl.ds(h * n_nodes, n_nodes), :] = jnp.concatenate(
                [ph, ones_col], axis=1)
            wrow = w_ref[pl.ds(h * c_head, c_head), :]
            pt = jnp.dot(wrow, nft, preferred_element_type=jnp.float32)
            pt = pt + bht_ref[pl.ds(h * c_head, c_head), :]
            a2 = a2t_ref[pl.ds(h * c_head, c_head), :]
            t_rows.append(jnp.sum(pt * a2, axis=0, keepdims=True))
        for h in range(num_heads):
            t_rows.append(_ALPHA * t_rows[h])
        t_tiles = [jnp.broadcast_to(r, (8, n_nodes)) for r in t_rows]
        tt_ref[...] = jnp.concatenate(t_tiles, axis=0)

    maskf = adj_ref[...].astype(jnp.float32)
    ones_row = jnp.ones((1, n_nodes), dtype=jnp.float32)
    outs = []
    for h in range(num_heads):
        ph_blk = projh_ref[pl.ds(h * n_nodes + i * blk, blk), :]
        a1 = a1_ref[h:h + 1, :]
        s_blk = jnp.sum(ph_blk * a1, axis=1, keepdims=True)
        t8 = tt_ref[pl.ds(8 * h, 8), :]
        ta8 = tt_ref[pl.ds(8 * (num_heads + h), 8), :]
        mt = jnp.max(t8[0:1, :])
        m_i = jnp.maximum(s_blk + mt, 0.0)
        s1 = (s_blk - m_i).reshape(blk // 8, 8, 1)
        s2 = (_ALPHA * s_blk - m_i).reshape(blk // 8, 8, 1)
        za = s1 + t8[None]
        zb = s2 + ta8[None]
        e = jnp.exp(jnp.maximum(za, zb)).reshape(blk, n_nodes) * maskf
        ph_full = projh_ref[pl.ds(h * n_nodes, n_nodes), :]
        acc = jnp.dot(e, ph_full, preferred_element_type=jnp.float32)
        outs.append(acc[:, :c_head] / acc[:, c_head:c_head + 1])
    out_ref[...] = jnp.concatenate(outs, axis=1)


def kernel(node_feats, adj_matrix, W, b, a):
    batch, n_nodes, c_in = node_feats.shape
    num_heads, two_c = a.shape
    c_head = two_c // 2
    c_out = num_heads * c_head

    blk = min(1024, n_nodes)
    grid = (n_nodes // blk,)

    wh = jnp.transpose(W.reshape(num_heads, c_head, c_in), (0, 2, 1))
    wh = wh.reshape(num_heads * c_in, c_head)
    a1 = jnp.pad(a[:, :c_head], ((0, 0), (0, 1)))
    a2t = a[:, c_head:].reshape(num_heads * c_head, 1)
    bh = b.reshape(num_heads, c_head)
    bht = b.reshape(num_heads * c_head, 1)

    const = lambda i: (0, 0)
    call = pl.pallas_call(
        lambda *refs: _gat_block_kernel(
            *refs, num_heads=num_heads, c_head=c_head, c_in=c_in,
            n_nodes=n_nodes, blk=blk),
        grid=grid,
        in_specs=[
            pl.BlockSpec((n_nodes, c_in), const),
            pl.BlockSpec((c_in, n_nodes), const),
            pl.BlockSpec((blk, n_nodes), lambda i: (i, 0)),
            pl.BlockSpec((num_heads * c_in, c_head), const),
            pl.BlockSpec((num_heads * c_head, c_in), const),
            pl.BlockSpec((num_heads, c_head + 1), const),
            pl.BlockSpec((num_heads * c_head, 1), const),
            pl.BlockSpec((num_heads, c_head), const),
            pl.BlockSpec((num_heads * c_head, 1), const),
        ],
        out_specs=pl.BlockSpec((blk, c_out), lambda i: (i, 0)),
        out_shape=jax.ShapeDtypeStruct((n_nodes, c_out), jnp.float32),
        scratch_shapes=[
            pltpu.VMEM((num_heads * n_nodes, c_head + 1), jnp.float32),
            pltpu.VMEM((16 * num_heads, n_nodes), jnp.float32),
        ],
    )

    outs = []
    for bb in range(batch):
        nf = node_feats[bb]
        outs.append(call(nf, nf.T, adj_matrix[bb], wh, W, a1, a2t, bh, bht))
    return jnp.stack(outs, axis=0)


# speedup vs baseline: 1.1642x; 1.0634x over previous
import jax
import jax.numpy as jnp
from jax.experimental import pallas as pl
from jax.experimental.pallas import tpu as pltpu

_ALPHA = 0.2
_NEG = -9e15


def _gat_block_kernel(nf_ref, nft_ref, adj_ref, wh_ref, w_ref, a1_ref,
                      a2t_ref, bh_ref, bht_ref, out_ref, projh_ref, tt_ref,
                      *, num_heads, c_head, c_in, n_nodes, blk):
    i = pl.program_id(0)

    @pl.when(i == 0)
    def _init():
        nf = nf_ref[...]
        nft = nft_ref[...]
        ones_col = jnp.ones((n_nodes, 1), dtype=jnp.float32)
        t_rows = []
        for h in range(num_heads):
            wh = wh_ref[pl.ds(h * c_in, c_in), :]
            ph = jnp.dot(nf, wh, preferred_element_type=jnp.float32)
            ph = ph + bh_ref[h:h + 1, :]
            projh_ref[pl.ds(h * n_nodes, n_nodes), :] = jnp.concatenate(
                [ph, ones_col], axis=1)
            wrow = w_ref[pl.ds(h * c_head, c_head), :]
            pt = jnp.dot(wrow, nft, preferred_element_type=jnp.float32)
            pt = pt + bht_ref[pl.ds(h * c_head, c_head), :]
            a2 = a2t_ref[pl.ds(h * c_head, c_head), :]
            t_rows.append(jnp.sum(pt * a2, axis=0, keepdims=True))
        for h in range(num_heads):
            t_rows.append(_ALPHA * t_rows[h])
        t_tiles = [jnp.broadcast_to(r, (8, n_nodes)) for r in t_rows]
        tt_ref[...] = jnp.concatenate(t_tiles, axis=0)

    maskf = adj_ref[...].astype(jnp.float32)
    ones_row = jnp.ones((1, n_nodes), dtype=jnp.float32)
    outs = []
    for h in range(num_heads):
        ph_blk = projh_ref[pl.ds(h * n_nodes + i * blk, blk), :]
        a1 = a1_ref[h:h + 1, :]
        s_blk = jnp.sum(ph_blk * a1, axis=1, keepdims=True)
        t8 = tt_ref[pl.ds(8 * h, 8), :]
        ta8 = tt_ref[pl.ds(8 * (num_heads + h), 8), :]
        mt = jnp.max(t8[0:1, :])
        m_i = jnp.maximum(s_blk + mt, 0.0)
        s1 = (s_blk - m_i).reshape(blk // 8, 8, 1)
        s2 = (_ALPHA * s_blk - m_i).reshape(blk // 8, 8, 1)
        za = s1 + t8[None]
        zb = s2 + ta8[None]
        e = jnp.exp2(jnp.maximum(za, zb)).reshape(blk, n_nodes) * maskf
        ph_full = projh_ref[pl.ds(h * n_nodes, n_nodes), :]
        acc = jnp.dot(e, ph_full, preferred_element_type=jnp.float32)
        outs.append(acc[:, :c_head] / acc[:, c_head:c_head + 1])
    out_ref[...] = jnp.concatenate(outs, axis=1)


def kernel(node_feats, adj_matrix, W, b, a):
    batch, n_nodes, c_in = node_feats.shape
    num_heads, two_c = a.shape
    c_head = two_c // 2
    c_out = num_heads * c_head

    blk = min(1024, n_nodes)
    grid = (n_nodes // blk,)

    wh = jnp.transpose(W.reshape(num_heads, c_head, c_in), (0, 2, 1))
    wh = wh.reshape(num_heads * c_in, c_head)
    log2e = 1.4426950408889634
    a1 = jnp.pad(a[:, :c_head] * log2e, ((0, 0), (0, 1)))
    a2t = (a[:, c_head:] * log2e).reshape(num_heads * c_head, 1)
    bh = b.reshape(num_heads, c_head)
    bht = b.reshape(num_heads * c_head, 1)

    const = lambda i: (0, 0)
    call = pl.pallas_call(
        lambda *refs: _gat_block_kernel(
            *refs, num_heads=num_heads, c_head=c_head, c_in=c_in,
            n_nodes=n_nodes, blk=blk),
        grid=grid,
        in_specs=[
            pl.BlockSpec((n_nodes, c_in), const),
            pl.BlockSpec((c_in, n_nodes), const),
            pl.BlockSpec((blk, n_nodes), lambda i: (i, 0)),
            pl.BlockSpec((num_heads * c_in, c_head), const),
            pl.BlockSpec((num_heads * c_head, c_in), const),
            pl.BlockSpec((num_heads, c_head + 1), const),
            pl.BlockSpec((num_heads * c_head, 1), const),
            pl.BlockSpec((num_heads, c_head), const),
            pl.BlockSpec((num_heads * c_head, 1), const),
        ],
        out_specs=pl.BlockSpec((blk, c_out), lambda i: (i, 0)),
        out_shape=jax.ShapeDtypeStruct((n_nodes, c_out), jnp.float32),
        scratch_shapes=[
            pltpu.VMEM((num_heads * n_nodes, c_head + 1), jnp.float32),
            pltpu.VMEM((16 * num_heads, n_nodes), jnp.float32),
        ],
    )

    outs = []
    for bb in range(batch):
        nf = node_feats[bb]
        outs.append(call(nf, nf.T, adj_matrix[bb], wh, W, a1, a2t, bh, bht))
    return jnp.stack(outs, axis=0)
